# Initial kernel scaffold; baseline (speedup 1.0000x reference)
import jax
import jax.numpy as jnp
from jax.experimental import pallas as pl


def kernel(x, A1, A2, A3, neighbor_index, neighbor_dist):
    raise NotImplementedError("write your pallas kernel here")


# speedup vs baseline: 121.7622x; 121.7622x over previous
import functools

import jax
import jax.numpy as jnp
from jax import lax
from jax.experimental import pallas as pl
from jax.experimental.pallas import tpu as pltpu
from jax.experimental.pallas import tpu_sc as plsc

_SIZES = (2048, 2048)
_N = _SIZES[0] * _SIZES[1]
_K = 8
_L = 16
_NC, _NS = 2, 16
_NW = _NC * _NS
_ROWS_W = _N // _NW
_B = 2048
_NB = _ROWS_W // _B


def _mm_body(a1_ref, a2_ref, a3_ref, g_ref):
    a23 = jnp.dot(a2_ref[...], a3_ref[...], preferred_element_type=jnp.float32)
    g_ref[...] = jnp.dot(a1_ref[...], a23, preferred_element_type=jnp.float32)


def _gather_body(table, idx_hbm, dist_hbm, out_hbm, idx_v, dst_v, gat_v, out_v, sem):
    wid = lax.axis_index("s") * _NC + lax.axis_index("c")
    lanes = lax.iota(jnp.int32, _L)
    p4 = lanes ^ 4
    p2 = lanes ^ 2
    p1 = lanes ^ 1
    pick = (lanes & 1) * 8
    half = lanes >> 1

    def batch(g, carry):
        row0 = wid * _ROWS_W + g * _B
        e0 = row0 * _K
        pltpu.sync_copy(idx_hbm.at[pl.ds(e0, _B * _K)], idx_v)
        pltpu.sync_copy(dist_hbm.at[pl.ds(e0, _B * _K)], dst_v)
        pltpu.async_copy(table.at[idx_v], gat_v, sem).wait()

        def outer(i, c):
            acc = jnp.zeros((_L,), jnp.float32)
            for j in range(_L // 2):
                off = (i * (_L // 2) + j) * _L
                v = gat_v[pl.ds(off, _L)] * dst_v[pl.ds(off, _L)]
                v = v + v[p4]
                v = v + v[p2]
                v = v + v[p1]
                acc = jnp.where(half == j, v[pick], acc)
            out_v[pl.ds(i * _L, _L)] = acc
            return c

        lax.fori_loop(0, _B // _L, outer, 0)
        pltpu.sync_copy(out_v, out_hbm.at[pl.ds(row0, _B)])
        return carry

    lax.fori_loop(0, _NB, batch, 0)


@jax.jit
def _run(A1, A2, A3, idx_flat, dist_flat):
    g = pl.pallas_call(
        _mm_body,
        out_shape=jax.ShapeDtypeStruct(_SIZES, jnp.float32),
    )(A1, A2, A3)
    table = g.reshape(_N)
    sc_gather = pl.kernel(
        _gather_body,
        out_type=jax.ShapeDtypeStruct((_N,), jnp.float32),
        mesh=plsc.VectorSubcoreMesh(
            core_axis_name="c", subcore_axis_name="s",
            num_cores=_NC, num_subcores=_NS,
        ),
        scratch_types=[
            pltpu.VMEM((_B * _K,), jnp.int32),
            pltpu.VMEM((_B * _K,), jnp.float32),
            pltpu.VMEM((_B * _K,), jnp.float32),
            pltpu.VMEM((_B,), jnp.float32),
            pltpu.SemaphoreType.DMA,
        ],
    )
    return sc_gather(table, idx_flat, dist_flat)


def kernel(x, A1, A2, A3, neighbor_index, neighbor_dist):
    idx_flat = neighbor_index.reshape(_N * _K)
    dist_flat = neighbor_dist.reshape(_N * _K)
    out_flat = _run(A1, A2, A3, idx_flat, dist_flat)
    return out_flat.reshape(_SIZES)


# speedup vs baseline: 130.2792x; 1.0699x over previous
import jax
import jax.numpy as jnp
from jax import lax
from jax.experimental import pallas as pl
from jax.experimental.pallas import tpu as pltpu
from jax.experimental.pallas import tpu_sc as plsc

_SIZES = (2048, 2048)
_N = _SIZES[0] * _SIZES[1]
_K = 8
_L = 16
_NC, _NS = 2, 16
_NW = _NC * _NS
_ROWS_W = _N // _NW
_B = 2048
_NB = _ROWS_W // _B
_E = _B * _K


def _mm_body(a1_ref, a2_ref, a3_ref, g_ref):
    a23 = jnp.dot(a2_ref[...], a3_ref[...], preferred_element_type=jnp.float32)
    g_ref[...] = jnp.dot(a1_ref[...], a23, preferred_element_type=jnp.float32)


def _gather_body(table, idx_hbm, dist_hbm, out_hbm,
                 idx0, idx1, dst0, dst1, gat0, gat1, out0, out1,
                 sl0, sl1, sg0, sg1, so0, so1):
    idxs = [idx0, idx1]
    dsts = [dst0, dst1]
    gats = [gat0, gat1]
    outs = [out0, out1]
    sls = [sl0, sl1]
    sgs = [sg0, sg1]
    sos = [so0, so1]

    wid = lax.axis_index("s") * _NC + lax.axis_index("c")
    base_e = wid * (_ROWS_W * _K)
    base_r = wid * _ROWS_W

    lanes = lax.iota(jnp.int32, _L)
    p4 = lanes ^ 4
    p2 = lanes ^ 2
    p1 = lanes ^ 1
    pick = (lanes & 1) * 8
    half = lanes >> 1

    def start_loads(g, b):
        e0 = base_e + g * _E
        pltpu.make_async_copy(idx_hbm.at[pl.ds(e0, _E)], idxs[b], sls[b]).start()
        pltpu.make_async_copy(dist_hbm.at[pl.ds(e0, _E)], dsts[b], sls[b]).start()

    def wait_loads(b):
        pltpu.make_async_copy(idx_hbm.at[pl.ds(0, _E)], idxs[b], sls[b]).wait()
        pltpu.make_async_copy(dist_hbm.at[pl.ds(0, _E)], dsts[b], sls[b]).wait()

    start_loads(0, 0)
    start_loads(1, 1)
    wait_loads(0)
    pltpu.make_async_copy(table.at[idxs[0]], gats[0], sgs[0]).start()

    def outer(gg, carry):
        for b in range(2):
            g = gg * 2 + b
            nb = 1 - b

            @pl.when(g + 1 < _NB)
            def _():
                wait_loads(nb)
                pltpu.make_async_copy(table.at[idxs[nb]], gats[nb], sgs[nb]).start()

            pltpu.make_async_copy(table.at[idxs[b]], gats[b], sgs[b]).wait()

            @pl.when(g + 2 < _NB)
            def _():
                e2 = base_e + (g + 2) * _E
                pltpu.make_async_copy(idx_hbm.at[pl.ds(e2, _E)], idxs[b], sls[b]).start()

            @pl.when(g >= 2)
            def _():
                pltpu.make_async_copy(outs[b], out_hbm.at[pl.ds(0, _B)], sos[b]).wait()

            def inner(i, c):
                acc = jnp.zeros((_L,), jnp.float32)
                for j in range(_L // 2):
                    off = (i * (_L // 2) + j) * _L
                    v = gats[b][pl.ds(off, _L)] * dsts[b][pl.ds(off, _L)]
                    v = v + v[p4]
                    v = v + v[p2]
                    v = v + v[p1]
                    acc = jnp.where(half == j, v[pick], acc)
                outs[b][pl.ds(i * _L, _L)] = acc
                return c

            lax.fori_loop(0, _B // _L, inner, 0)
            pltpu.make_async_copy(
                outs[b], out_hbm.at[pl.ds(base_r + g * _B, _B)], sos[b]).start()

            @pl.when(g + 2 < _NB)
            def _():
                e2 = base_e + (g + 2) * _E
                pltpu.make_async_copy(dist_hbm.at[pl.ds(e2, _E)], dsts[b], sls[b]).start()

        return carry

    lax.fori_loop(0, _NB // 2, outer, 0)
    for b in range(2):
        pltpu.make_async_copy(outs[b], out_hbm.at[pl.ds(0, _B)], sos[b]).wait()


@jax.jit
def _run(A1, A2, A3, idx_flat, dist_flat):
    g = pl.pallas_call(
        _mm_body,
        out_shape=jax.ShapeDtypeStruct(_SIZES, jnp.float32),
    )(A1, A2, A3)
    table = g.reshape(_N)
    sc_gather = pl.kernel(
        _gather_body,
        out_type=jax.ShapeDtypeStruct((_N,), jnp.float32),
        mesh=plsc.VectorSubcoreMesh(
            core_axis_name="c", subcore_axis_name="s",
            num_cores=_NC, num_subcores=_NS,
        ),
        scratch_types=[
            pltpu.VMEM((_E,), jnp.int32), pltpu.VMEM((_E,), jnp.int32),
            pltpu.VMEM((_E,), jnp.float32), pltpu.VMEM((_E,), jnp.float32),
            pltpu.VMEM((_E,), jnp.float32), pltpu.VMEM((_E,), jnp.float32),
            pltpu.VMEM((_B,), jnp.float32), pltpu.VMEM((_B,), jnp.float32),
            pltpu.SemaphoreType.DMA, pltpu.SemaphoreType.DMA,
            pltpu.SemaphoreType.DMA, pltpu.SemaphoreType.DMA,
            pltpu.SemaphoreType.DMA, pltpu.SemaphoreType.DMA,
        ],
    )
    return sc_gather(table, idx_flat, dist_flat)


def kernel(x, A1, A2, A3, neighbor_index, neighbor_dist):
    idx_flat = neighbor_index.reshape(_N * _K)
    dist_flat = neighbor_dist.reshape(_N * _K)
    out_flat = _run(A1, A2, A3, idx_flat, dist_flat)
    return out_flat.reshape(_SIZES)


# speedup vs baseline: 427.0235x; 3.2778x over previous
import jax
import jax.numpy as jnp
from jax import lax
from jax.experimental import pallas as pl
from jax.experimental.pallas import tpu as pltpu
from jax.experimental.pallas import tpu_sc as plsc

_SIZES = (2048, 2048)
_N = _SIZES[0] * _SIZES[1]
_K = 8
_L = 16
_NC, _NS = 2, 16
_NW = _NC * _NS
_ROWS_W = _N // _NW
_B = 2048
_NB = _ROWS_W // _B
_E = _B * _K
_TB = _B // 128


def _mm_body(a1_ref, a2_ref, a3_ref, g_ref):
    a23 = jnp.dot(a2_ref[...], a3_ref[...], preferred_element_type=jnp.float32)
    g_ref[...] = jnp.dot(a1_ref[...], a23, preferred_element_type=jnp.float32)


def _gather_body(table, idx_hbm, dist_hbm, out_hbm,
                 idx0, idx1, dst0, dst1, gat0, gat1, out0, out1,
                 sl0, sl1, sg0, sg1, so0, so1):
    idxs = [idx0, idx1]
    dsts = [dst0, dst1]
    gats = [gat0, gat1]
    outs = [out0, out1]
    sls = [sl0, sl1]
    sgs = [sg0, sg1]
    sos = [so0, so1]

    wid = lax.axis_index("s") * _NC + lax.axis_index("c")
    base_r = wid * _ROWS_W

    def start_loads(g, b):
        e0 = (base_r + g * _B) * _K
        pltpu.make_async_copy(idx_hbm.at[pl.ds(e0, _E)], idxs[b], sls[b]).start()
        r0 = base_r + g * _B
        for k in range(_K):
            pltpu.make_async_copy(
                dist_hbm.at[pl.ds(k * _N + r0, _B)],
                dsts[b].at[pl.ds(k * _B, _B)], sls[b]).start()

    def wait_loads(b):
        pltpu.make_async_copy(idx_hbm.at[pl.ds(0, _E)], idxs[b], sls[b]).wait()
        for k in range(_K):
            pltpu.make_async_copy(
                dist_hbm.at[pl.ds(0, _B)],
                dsts[b].at[pl.ds(k * _B, _B)], sls[b]).wait()

    start_loads(0, 0)
    start_loads(1, 1)
    wait_loads(0)
    pltpu.make_async_copy(table.at[idxs[0]], gats[0], sgs[0]).start()

    def outer(gg, carry):
        for b in range(2):
            g = gg * 2 + b
            nb = 1 - b

            @pl.when(g + 1 < _NB)
            def _():
                wait_loads(nb)
                pltpu.make_async_copy(table.at[idxs[nb]], gats[nb], sgs[nb]).start()

            pltpu.make_async_copy(table.at[idxs[b]], gats[b], sgs[b]).wait()

            @pl.when(g + 2 < _NB)
            def _():
                e2 = (base_r + (g + 2) * _B) * _K
                pltpu.make_async_copy(idx_hbm.at[pl.ds(e2, _E)], idxs[b], sls[b]).start()

            @pl.when(g >= 2)
            def _():
                pltpu.make_async_copy(outs[b], out_hbm.at[pl.ds(0, _B)], sos[b]).wait()

            def inner(tb, c):
                gbase = tb * (128 * _K)
                for jj in range(8):
                    acc = jnp.zeros((_L,), jnp.float32)
                    for k in range(_K):
                        gv = gats[b][pl.ds(gbase + k * 128 + jj * _L, _L)]
                        wv = dsts[b][pl.ds(k * _B + tb * 128 + jj * _L, _L)]
                        acc = acc + gv * wv
                    outs[b][pl.ds(tb * 128 + jj * _L, _L)] = acc
                return c

            lax.fori_loop(0, _TB, inner, 0)
            pltpu.make_async_copy(
                outs[b], out_hbm.at[pl.ds(base_r + g * _B, _B)], sos[b]).start()

            @pl.when(g + 2 < _NB)
            def _():
                r2 = base_r + (g + 2) * _B
                for k in range(_K):
                    pltpu.make_async_copy(
                        dist_hbm.at[pl.ds(k * _N + r2, _B)],
                        dsts[b].at[pl.ds(k * _B, _B)], sls[b]).start()

        return carry

    lax.fori_loop(0, _NB // 2, outer, 0)
    for b in range(2):
        pltpu.make_async_copy(outs[b], out_hbm.at[pl.ds(0, _B)], sos[b]).wait()


@jax.jit
def _run(A1, A2, A3, idx_t, dist_t):
    g = pl.pallas_call(
        _mm_body,
        out_shape=jax.ShapeDtypeStruct(_SIZES, jnp.float32),
    )(A1, A2, A3)
    table = g.reshape(_N)
    sc_gather = pl.kernel(
        _gather_body,
        out_type=jax.ShapeDtypeStruct((_N,), jnp.float32),
        mesh=plsc.VectorSubcoreMesh(
            core_axis_name="c", subcore_axis_name="s",
            num_cores=_NC, num_subcores=_NS,
        ),
        scratch_types=[
            pltpu.VMEM((_E,), jnp.int32), pltpu.VMEM((_E,), jnp.int32),
            pltpu.VMEM((_E,), jnp.float32), pltpu.VMEM((_E,), jnp.float32),
            pltpu.VMEM((_E,), jnp.float32), pltpu.VMEM((_E,), jnp.float32),
            pltpu.VMEM((_B,), jnp.float32), pltpu.VMEM((_B,), jnp.float32),
            pltpu.SemaphoreType.DMA, pltpu.SemaphoreType.DMA,
            pltpu.SemaphoreType.DMA, pltpu.SemaphoreType.DMA,
            pltpu.SemaphoreType.DMA, pltpu.SemaphoreType.DMA,
        ],
    )
    return sc_gather(table, idx_t, dist_t)


def kernel(x, A1, A2, A3, neighbor_index, neighbor_dist):
    idx_t = neighbor_index.reshape(_N // 128, 128, _K).transpose(0, 2, 1).reshape(_N * _K)
    dist_t = neighbor_dist.transpose(1, 2, 0).reshape(_N * _K)
    out_flat = _run(A1, A2, A3, idx_t, dist_t)
    return out_flat.reshape(_SIZES)


# speedup vs baseline: 432.8753x; 1.0137x over previous
import jax
import jax.numpy as jnp
from jax import lax
from jax.experimental import pallas as pl
from jax.experimental.pallas import tpu as pltpu
from jax.experimental.pallas import tpu_sc as plsc

_SIZES = (2048, 2048)
_N = _SIZES[0] * _SIZES[1]
_K = 8
_L = 16
_NC, _NS = 2, 16
_NW = _NC * _NS
_ROWS_W = _N // _NW
_B = 2048
_NB = _ROWS_W // _B
_E = _B * _K
_TB = _B // 128


def _mm_body(a1_ref, a2_ref, a3_ref, g_ref):
    a23 = jnp.dot(a2_ref[...], a3_ref[...], preferred_element_type=jnp.float32)
    g_ref[...] = jnp.dot(a1_ref[...], a23, preferred_element_type=jnp.float32)


def _gather_body(table, idx_hbm, dist_hbm, out_hbm,
                 idx0, idx1, dst0, dst1, gat0, gat1, out0, out1,
                 sl0, sl1, sg0, sg1, so0, so1):
    idxs = [idx0, idx1]
    dsts = [dst0, dst1]
    gats = [gat0, gat1]
    outs = [out0, out1]
    sls = [sl0, sl1]
    sgs = [sg0, sg1]
    sos = [so0, so1]

    wid = lax.axis_index("s") * _NC + lax.axis_index("c")
    base_r = wid * _ROWS_W

    def start_loads(g, b):
        e0 = (base_r + g * _B) * _K
        pltpu.make_async_copy(idx_hbm.at[pl.ds(e0, _E)], idxs[b], sls[b]).start()
        r0 = base_r + g * _B
        for k in range(_K):
            pltpu.make_async_copy(
                dist_hbm.at[pl.ds(k * _N + r0, _B)],
                dsts[b].at[pl.ds(k * _B, _B)], sls[b]).start()

    def wait_loads(b):
        pltpu.make_async_copy(idx_hbm.at[pl.ds(0, _E)], idxs[b], sls[b]).wait()
        for k in range(_K):
            pltpu.make_async_copy(
                dist_hbm.at[pl.ds(0, _B)],
                dsts[b].at[pl.ds(k * _B, _B)], sls[b]).wait()

    start_loads(0, 0)
    start_loads(1, 1)
    wait_loads(0)
    pltpu.make_async_copy(table.at[idxs[0]], gats[0], sgs[0]).start()

    def outer(gg, carry):
        for b in range(2):
            g = gg * 2 + b
            nb = 1 - b

            @pl.when(g + 1 < _NB)
            def _():
                wait_loads(nb)
                pltpu.make_async_copy(table.at[idxs[nb]], gats[nb], sgs[nb]).start()

            pltpu.make_async_copy(table.at[idxs[b]], gats[b], sgs[b]).wait()

            @pl.when(g + 2 < _NB)
            def _():
                e2 = (base_r + (g + 2) * _B) * _K
                pltpu.make_async_copy(idx_hbm.at[pl.ds(e2, _E)], idxs[b], sls[b]).start()

            @pl.when(g >= 2)
            def _():
                pltpu.make_async_copy(outs[b], out_hbm.at[0], sos[b]).wait()

            def inner(tb, c):
                gbase = tb * (128 * _K)
                for jj in range(8):
                    acc = jnp.zeros((_L,), jnp.float32)
                    for k in range(_K):
                        gv = gats[b][pl.ds(gbase + k * 128 + jj * _L, _L)]
                        wv = dsts[b][pl.ds(k * _B + tb * 128 + jj * _L, _L)]
                        acc = acc + gv * wv
                    outs[b][pl.ds(tb * 128 + jj * _L, _L)] = acc
                return c

            lax.fori_loop(0, _TB, inner, 0)
            pltpu.make_async_copy(
                outs[b], out_hbm.at[wid * _NB + g], sos[b]).start()

            @pl.when(g + 2 < _NB)
            def _():
                r2 = base_r + (g + 2) * _B
                for k in range(_K):
                    pltpu.make_async_copy(
                        dist_hbm.at[pl.ds(k * _N + r2, _B)],
                        dsts[b].at[pl.ds(k * _B, _B)], sls[b]).start()

        return carry

    lax.fori_loop(0, _NB // 2, outer, 0)
    for b in range(2):
        pltpu.make_async_copy(outs[b], out_hbm.at[0], sos[b]).wait()


@jax.jit
def _run(A1, A2, A3, idx_t, dist_t):
    g = pl.pallas_call(
        _mm_body,
        out_shape=jax.ShapeDtypeStruct(_SIZES, jnp.float32),
    )(A1, A2, A3)
    table = g.reshape(_N)
    sc_gather = pl.kernel(
        _gather_body,
        out_type=jax.ShapeDtypeStruct(_SIZES, jnp.float32),
        mesh=plsc.VectorSubcoreMesh(
            core_axis_name="c", subcore_axis_name="s",
            num_cores=_NC, num_subcores=_NS,
        ),
        scratch_types=[
            pltpu.VMEM((_E,), jnp.int32), pltpu.VMEM((_E,), jnp.int32),
            pltpu.VMEM((_E,), jnp.float32), pltpu.VMEM((_E,), jnp.float32),
            pltpu.VMEM((_E,), jnp.float32), pltpu.VMEM((_E,), jnp.float32),
            pltpu.VMEM((_B,), jnp.float32), pltpu.VMEM((_B,), jnp.float32),
            pltpu.SemaphoreType.DMA, pltpu.SemaphoreType.DMA,
            pltpu.SemaphoreType.DMA, pltpu.SemaphoreType.DMA,
            pltpu.SemaphoreType.DMA, pltpu.SemaphoreType.DMA,
        ],
    )
    return sc_gather(table, idx_t, dist_t)


def kernel(x, A1, A2, A3, neighbor_index, neighbor_dist):
    idx_t = neighbor_index.reshape(_N // 128, 128, _K).transpose(0, 2, 1).reshape(_N * _K)
    dist_t = neighbor_dist.transpose(1, 2, 0).reshape(_N * _K)
    return _run(A1, A2, A3, idx_t, dist_t)
